# baseline (speedup 1.0000x reference)
import functools

import jax
import jax.numpy as jnp
from jax import lax
from jax.experimental import pallas as pl
from jax.experimental.pallas import tpu as pltpu
from jax.experimental.pallas import tpu_sc as plsc

_V = 1000
_N = 204800
_NC, _NS = 2, 16
_NW = _NC * _NS
_PER_W = _N // _NW
_NCG = _V // 8
_NTI = _N // 128
_TPW = _NTI // _NW
_HT = _TPW // 2


def _lse_body(t_ref, o_ref):
    t = t_ref[...]
    m = jnp.max(t, axis=1, keepdims=True)
    o_ref[...] = m + jnp.log(jnp.sum(jnp.exp(t - m), axis=1, keepdims=True))


_lse_call = pl.pallas_call(
    _lse_body,
    out_shape=jax.ShapeDtypeStruct((_V, 1), jnp.float32),
)


def _loss_body(p_ref, o_ref):
    o_ref[0, 0] = jnp.sum(p_ref[...]) * (1.0 / _N)


_loss_call = pl.pallas_call(
    _loss_body,
    out_shape=jax.ShapeDtypeStruct((1, 1), jnp.float32),
    out_specs=pl.BlockSpec(memory_space=pltpu.SMEM),
)


_mesh = plsc.VectorSubcoreMesh(core_axis_name="c", subcore_axis_name="s")


@functools.partial(
    pl.kernel,
    out_type=(
        jax.ShapeDtypeStruct((_NCG, _NTI, 8, 128), jnp.float32),
        jax.ShapeDtypeStruct((_NW, 16), jnp.float32),
    ),
    mesh=_mesh,
    compiler_params=pltpu.CompilerParams(
        use_tc_tiling_on_sc=False, needs_layout_passes=False),
    scratch_types=[
        pltpu.VMEM((_PER_W,), jnp.int32),
        pltpu.VMEM((_PER_W,), jnp.int32),
        pltpu.VMEM((_PER_W,), jnp.float32),
        pltpu.VMEM((_V,), jnp.float32),
        pltpu.VMEM((16 * _V,), jnp.float32),
        pltpu.VMEM((_HT, 8, 128), jnp.float32),
        pltpu.VMEM((_HT, 8, 128), jnp.float32),
        pltpu.VMEM((16,), jnp.float32),
        pltpu.SemaphoreType.DMA,
        pltpu.SemaphoreType.DMA,
        pltpu.SemaphoreType.DMA,
        pltpu.SemaphoreType.DMA,
        pltpu.SemaphoreType.DMA,
    ],
)
def _sc_main(tt_hbm, idx_hbm, tgt_hbm, lse_hbm, out_hbm, part_hbm,
             idx_v, off_v, picked_v, lse_v, tt_v, st_a, st_b,
             acc_v, sem_ta, sem_tb, sem_sa, sem_sb, sem_p):
    wid = lax.axis_index("s") * _NC + lax.axis_index("c")
    ebase = wid * _PER_W
    tbase = wid * _TPW

    pltpu.sync_copy(idx_hbm.at[pl.ds(ebase, _PER_W)], idx_v)
    pltpu.sync_copy(tgt_hbm.at[pl.ds(ebase, _PER_W)], off_v)
    pltpu.sync_copy(lse_hbm, lse_v)
    acc_v[...] = jnp.zeros((16,), jnp.float32)

    def off_body(g, carry):
        sl = pl.ds(g * 16, 16)
        off_v[sl] = off_v[sl] * _V + idx_v[sl]
        return carry

    lax.fori_loop(0, _PER_W // 16, off_body, 0)

    for c in range(_PER_W // 128):
        pltpu.async_copy(
            tt_hbm.at[off_v.at[pl.ds(c * 128, 128)]],
            picked_v.at[pl.ds(c * 128, 128)],
            sem_p,
        )
    for c in range(_PER_W // 128):
        pltpu.make_async_copy(
            tt_hbm.at[off_v.at[pl.ds(c * 128, 128)]],
            picked_v.at[pl.ds(c * 128, 128)],
            sem_p,
        ).wait()

    def lp_body(g, carry):
        sl = pl.ds(g * 16, 16)
        lseg = plsc.load_gather(lse_v, [idx_v[sl]])
        acc_v[...] = acc_v[...] + (lseg - picked_v[sl])
        return carry

    lax.fori_loop(0, _PER_W // 16, lp_body, 0)
    pltpu.sync_copy(acc_v, part_hbm.at[wid])

    def tt_start(cg, half, sem):
        pltpu.async_copy(
            tt_hbm.at[pl.ds(cg * (8 * _V), 8 * _V)],
            tt_v.at[pl.ds(half * (8 * _V), 8 * _V)], sem)

    def tt_wait(cg, half, sem):
        pltpu.make_async_copy(
            tt_hbm.at[pl.ds(cg * (8 * _V), 8 * _V)],
            tt_v.at[pl.ds(half * (8 * _V), 8 * _V)], sem).wait()

    def st_start(cg, h, st, sem):
        pltpu.async_copy(st, out_hbm.at[cg, pl.ds(tbase + h * _HT, _HT)], sem)

    def st_wait(cg, h, st, sem):
        pltpu.make_async_copy(
            st, out_hbm.at[cg, pl.ds(tbase + h * _HT, _HT)], sem).wait()

    def do_group(pofs, h, st):
        @plsc.parallel_loop(0, _HT, unroll=5)
        def _(ip):
            e0 = h * (_HT * 128) + ip * 128
            ivs = [idx_v[pl.ds(e0 + m * 16, 16)] + pofs for m in range(8)]
            for s in range(8):
                for m in range(8):
                    st[ip, s, pl.ds(m * 16, 16)] = plsc.load_gather(
                        tt_v, [ivs[m] + (s * _V)])

    tt_start(0, 0, sem_ta)

    def c_body(cg, carry):
        p = cg % 2

        @pl.when(p == 0)
        def _():
            tt_wait(cg, 0, sem_ta)

            @pl.when(cg + 1 < _NCG)
            def _():
                tt_start(cg + 1, 1, sem_tb)

        @pl.when(p == 1)
        def _():
            tt_wait(cg, 1, sem_tb)

            @pl.when(cg + 1 < _NCG)
            def _():
                tt_start(cg + 1, 0, sem_ta)

        pofs = p * (8 * _V)

        @pl.when(cg > 0)
        def _():
            st_wait(cg - 1, 0, st_a, sem_sa)

        do_group(pofs, 0, st_a)
        st_start(cg, 0, st_a, sem_sa)

        @pl.when(cg > 0)
        def _():
            st_wait(cg - 1, 1, st_b, sem_sb)

        do_group(pofs, 1, st_b)
        st_start(cg, 1, st_b, sem_sb)
        return carry

    lax.fori_loop(0, _NCG, c_body, 0)
    st_wait(_NCG - 1, 0, st_a, sem_sa)
    st_wait(_NCG - 1, 1, st_b, sem_sb)


def kernel(idx, targets, token_emb_table):
    tableT_flat = token_emb_table.T.reshape(_V * _V)
    idx_f = idx.reshape(_N)
    tgt_f = targets.reshape(_N)
    lse = _lse_call(token_emb_table).reshape(_V)
    y4, partials = _sc_main(tableT_flat, idx_f, tgt_f, lse)
    logits2 = y4.transpose(0, 2, 1, 3).reshape(_V, _N).T
    loss = _loss_call(partials)[0, 0]
    return (logits2, loss)


# speedup vs baseline: 1.7891x; 1.7891x over previous
import functools

import jax
import jax.numpy as jnp
from jax import lax
from jax.experimental import pallas as pl
from jax.experimental.pallas import tpu as pltpu
from jax.experimental.pallas import tpu_sc as plsc

_V = 1000
_N = 204800
_NC, _NS = 2, 16
_NW = _NC * _NS
_PER_W = _N // _NW
_NCG = _V // 8
_NTI = _N // 128
_TPW = _NTI // _NW
_HT = _TPW // 2


def _lse_body(t_ref, o_ref):
    t = t_ref[...]
    m = jnp.max(t, axis=1, keepdims=True)
    o_ref[...] = m + jnp.log(jnp.sum(jnp.exp(t - m), axis=1, keepdims=True))


_lse_call = pl.pallas_call(
    _lse_body,
    out_shape=jax.ShapeDtypeStruct((_V, 1), jnp.float32),
)


def _loss_body(p_ref, o_ref):
    o_ref[0, 0] = jnp.sum(p_ref[...]) * (1.0 / _N)


_loss_call = pl.pallas_call(
    _loss_body,
    out_shape=jax.ShapeDtypeStruct((1, 1), jnp.float32),
    out_specs=pl.BlockSpec(memory_space=pltpu.SMEM),
)


_mesh = plsc.VectorSubcoreMesh(core_axis_name="c", subcore_axis_name="s")


@functools.partial(
    pl.kernel,
    out_type=(
        jax.ShapeDtypeStruct((_NCG, _NTI, 8, 128), jnp.float32),
        jax.ShapeDtypeStruct((_NW, 16), jnp.float32),
    ),
    mesh=_mesh,
    compiler_params=pltpu.CompilerParams(
        use_tc_tiling_on_sc=False, needs_layout_passes=False),
    scratch_types=[
        pltpu.VMEM((_PER_W,), jnp.int32),
        pltpu.VMEM((_PER_W,), jnp.int32),
        pltpu.VMEM((_PER_W,), jnp.float32),
        pltpu.VMEM((_V,), jnp.float32),
        pltpu.VMEM((8 * _V,), jnp.float32),
        pltpu.VMEM((8 * _V,), jnp.float32),
        pltpu.VMEM((_HT, 8, 128), jnp.float32),
        pltpu.VMEM((_HT, 8, 128), jnp.float32),
        pltpu.VMEM((16,), jnp.float32),
        pltpu.SemaphoreType.DMA,
        pltpu.SemaphoreType.DMA,
        pltpu.SemaphoreType.DMA,
        pltpu.SemaphoreType.DMA,
        pltpu.SemaphoreType.DMA,
    ],
)
def _sc_main(tt_hbm, idx_hbm, tgt_hbm, lse_hbm, out_hbm, part_hbm,
             idx_v, off_v, picked_v, lse_v, tt_a, tt_b, st_a, st_b,
             acc_v, sem_ta, sem_tb, sem_sa, sem_sb, sem_p):
    wid = lax.axis_index("s") * _NC + lax.axis_index("c")
    ebase = wid * _PER_W
    tbase = wid * _TPW

    pltpu.sync_copy(idx_hbm.at[pl.ds(ebase, _PER_W)], idx_v)
    pltpu.sync_copy(tgt_hbm.at[pl.ds(ebase, _PER_W)], off_v)
    pltpu.sync_copy(lse_hbm, lse_v)
    acc_v[...] = jnp.zeros((16,), jnp.float32)

    def off_body(g, carry):
        sl = pl.ds(g * 16, 16)
        off_v[sl] = off_v[sl] * _V + idx_v[sl]
        return carry

    lax.fori_loop(0, _PER_W // 16, off_body, 0)

    for c in range(_PER_W // 128):
        pltpu.async_copy(
            tt_hbm.at[off_v.at[pl.ds(c * 128, 128)]],
            picked_v.at[pl.ds(c * 128, 128)],
            sem_p,
        )
    for c in range(_PER_W // 128):
        pltpu.make_async_copy(
            tt_hbm.at[off_v.at[pl.ds(c * 128, 128)]],
            picked_v.at[pl.ds(c * 128, 128)],
            sem_p,
        ).wait()

    def lp_body(g, carry):
        sl = pl.ds(g * 16, 16)
        lseg = plsc.load_gather(lse_v, [idx_v[sl]])
        acc_v[...] = acc_v[...] + (lseg - picked_v[sl])
        return carry

    lax.fori_loop(0, _PER_W // 16, lp_body, 0)
    pltpu.sync_copy(acc_v, part_hbm.at[wid])

    def tt_start(cg, tt, sem):
        pltpu.async_copy(tt_hbm.at[pl.ds(cg * (8 * _V), 8 * _V)], tt, sem)

    def tt_wait(cg, tt, sem):
        pltpu.make_async_copy(
            tt_hbm.at[pl.ds(cg * (8 * _V), 8 * _V)], tt, sem).wait()

    def st_start(cg, h, st, sem):
        pltpu.async_copy(st, out_hbm.at[cg, pl.ds(tbase + h * _HT, _HT)], sem)

    def st_wait(cg, h, st, sem):
        pltpu.make_async_copy(
            st, out_hbm.at[cg, pl.ds(tbase + h * _HT, _HT)], sem).wait()

    def do_group(tt, h, st):
        @plsc.parallel_loop(0, _HT * 8, unroll=2)
        def _(t):
            ip = t // 8
            m = t % 8
            iv = idx_v[pl.ds(h * (_HT * 128) + t * 16, 16)]
            for s in range(8):
                st[ip, s, pl.ds(m * 16, 16)] = plsc.load_gather(
                    tt, [iv + (s * _V)])

    tt_start(0, tt_a, sem_ta)

    def pair_body(t, carry):
        c0 = 2 * t

        tt_wait(c0, tt_a, sem_ta)
        tt_start(c0 + 1, tt_b, sem_tb)

        @pl.when(t > 0)
        def _():
            st_wait(c0 - 1, 0, st_a, sem_sa)

        do_group(tt_a, 0, st_a)
        st_start(c0, 0, st_a, sem_sa)

        @pl.when(t > 0)
        def _():
            st_wait(c0 - 1, 1, st_b, sem_sb)

        do_group(tt_a, 1, st_b)
        st_start(c0, 1, st_b, sem_sb)

        tt_wait(c0 + 1, tt_b, sem_tb)
        tt_start(c0 + 2, tt_a, sem_ta)
        st_wait(c0, 0, st_a, sem_sa)
        do_group(tt_b, 0, st_a)
        st_start(c0 + 1, 0, st_a, sem_sa)
        st_wait(c0, 1, st_b, sem_sb)
        do_group(tt_b, 1, st_b)
        st_start(c0 + 1, 1, st_b, sem_sb)
        return carry

    lax.fori_loop(0, (_NCG - 1) // 2, pair_body, 0)

    cg = _NCG - 1
    tt_wait(cg, tt_a, sem_ta)
    st_wait(cg - 1, 0, st_a, sem_sa)
    do_group(tt_a, 0, st_a)
    st_start(cg, 0, st_a, sem_sa)
    st_wait(cg - 1, 1, st_b, sem_sb)
    do_group(tt_a, 1, st_b)
    st_start(cg, 1, st_b, sem_sb)
    st_wait(cg, 0, st_a, sem_sa)
    st_wait(cg, 1, st_b, sem_sb)


def kernel(idx, targets, token_emb_table):
    tableT_flat = token_emb_table.T.reshape(_V * _V)
    idx_f = idx.reshape(_N)
    tgt_f = targets.reshape(_N)
    lse = _lse_call(token_emb_table).reshape(_V)
    y4, partials = _sc_main(tableT_flat, idx_f, tgt_f, lse)
    logits2 = y4.transpose(0, 2, 1, 3).reshape(_V, _N).T
    loss = _loss_call(partials)[0, 0]
    return (logits2, loss)


# speedup vs baseline: 1.7931x; 1.0022x over previous
import functools

import jax
import jax.numpy as jnp
from jax import lax
from jax.experimental import pallas as pl
from jax.experimental.pallas import tpu as pltpu
from jax.experimental.pallas import tpu_sc as plsc

_V = 1000
_N = 204800
_NC, _NS = 2, 16
_NW = _NC * _NS
_PER_W = _N // _NW
_NCG = _V // 8
_NTI = _N // 128
_TPW = _NTI // _NW
_HT = _TPW // 2


def _lse_body(t_ref, o_ref):
    t = t_ref[...]
    m = jnp.max(t, axis=1, keepdims=True)
    o_ref[...] = m + jnp.log(jnp.sum(jnp.exp(t - m), axis=1, keepdims=True))


_lse_call = pl.pallas_call(
    _lse_body,
    out_shape=jax.ShapeDtypeStruct((_V, 1), jnp.float32),
)


def _loss_body(p_ref, o_ref):
    o_ref[0, 0] = jnp.sum(p_ref[...]) * (1.0 / _N)


_loss_call = pl.pallas_call(
    _loss_body,
    out_shape=jax.ShapeDtypeStruct((1, 1), jnp.float32),
    out_specs=pl.BlockSpec(memory_space=pltpu.SMEM),
)


_mesh = plsc.VectorSubcoreMesh(core_axis_name="c", subcore_axis_name="s")


@functools.partial(
    pl.kernel,
    out_type=(
        jax.ShapeDtypeStruct((_NCG, _NTI, 8, 128), jnp.float32),
        jax.ShapeDtypeStruct((_NW, 16), jnp.float32),
    ),
    mesh=_mesh,
    compiler_params=pltpu.CompilerParams(
        use_tc_tiling_on_sc=False, needs_layout_passes=False),
    scratch_types=[
        pltpu.VMEM((_PER_W,), jnp.int32),
        pltpu.VMEM((_PER_W,), jnp.int32),
        pltpu.VMEM((_PER_W,), jnp.float32),
        pltpu.VMEM((_V,), jnp.float32),
        pltpu.VMEM((8 * _V,), jnp.float32),
        pltpu.VMEM((8 * _V,), jnp.float32),
        pltpu.VMEM((_HT, 8, 128), jnp.float32),
        pltpu.VMEM((_HT, 8, 128), jnp.float32),
        pltpu.VMEM((16,), jnp.float32),
        pltpu.SemaphoreType.DMA,
        pltpu.SemaphoreType.DMA,
        pltpu.SemaphoreType.DMA,
        pltpu.SemaphoreType.DMA,
        pltpu.SemaphoreType.DMA,
    ],
)
def _sc_main(tt_hbm, idx_hbm, tgt_hbm, lse_hbm, out_hbm, part_hbm,
             idx_v, off_v, picked_v, lse_v, tt_a, tt_b, st_a, st_b,
             acc_v, sem_ta, sem_tb, sem_sa, sem_sb, sem_p):
    wid = lax.axis_index("s") * _NC + lax.axis_index("c")
    ebase = wid * _PER_W
    tbase = wid * _TPW

    pltpu.sync_copy(idx_hbm.at[pl.ds(ebase, _PER_W)], idx_v)
    pltpu.sync_copy(tgt_hbm.at[pl.ds(ebase, _PER_W)], off_v)
    pltpu.sync_copy(lse_hbm, lse_v)
    acc_v[...] = jnp.zeros((16,), jnp.float32)

    def off_body(g, carry):
        sl = pl.ds(g * 16, 16)
        off_v[sl] = off_v[sl] * _V + idx_v[sl]
        return carry

    lax.fori_loop(0, _PER_W // 16, off_body, 0)

    for c in range(_PER_W // 128):
        pltpu.async_copy(
            tt_hbm.at[off_v.at[pl.ds(c * 128, 128)]],
            picked_v.at[pl.ds(c * 128, 128)],
            sem_p,
        )
    for c in range(_PER_W // 128):
        pltpu.make_async_copy(
            tt_hbm.at[off_v.at[pl.ds(c * 128, 128)]],
            picked_v.at[pl.ds(c * 128, 128)],
            sem_p,
        ).wait()

    def lp_body(g, carry):
        sl = pl.ds(g * 16, 16)
        lseg = plsc.load_gather(lse_v, [idx_v[sl]])
        acc_v[...] = acc_v[...] + (lseg - picked_v[sl])
        return carry

    lax.fori_loop(0, _PER_W // 16, lp_body, 0)
    pltpu.sync_copy(acc_v, part_hbm.at[wid])

    def tt_start(cg, tt, sem):
        pltpu.async_copy(tt_hbm.at[pl.ds(cg * (8 * _V), 8 * _V)], tt, sem)

    def tt_wait(cg, tt, sem):
        pltpu.make_async_copy(
            tt_hbm.at[pl.ds(cg * (8 * _V), 8 * _V)], tt, sem).wait()

    def st_start(cg, h, st, sem):
        pltpu.async_copy(st, out_hbm.at[cg, pl.ds(tbase + h * _HT, _HT)], sem)

    def st_wait(cg, h, st, sem):
        pltpu.make_async_copy(
            st, out_hbm.at[cg, pl.ds(tbase + h * _HT, _HT)], sem).wait()

    def do_group(tt, h, st):
        @plsc.parallel_loop(0, _HT * 8, unroll=4)
        def _(t):
            ip = t // 8
            m = t % 8
            iv = idx_v[pl.ds(h * (_HT * 128) + t * 16, 16)]
            for s in range(8):
                st[ip, s, pl.ds(m * 16, 16)] = plsc.load_gather(
                    tt, [iv + (s * _V)])

    tt_start(0, tt_a, sem_ta)

    def pair_body(t, carry):
        c0 = 2 * t

        tt_wait(c0, tt_a, sem_ta)
        tt_start(c0 + 1, tt_b, sem_tb)

        @pl.when(t > 0)
        def _():
            st_wait(c0 - 1, 0, st_a, sem_sa)

        do_group(tt_a, 0, st_a)
        st_start(c0, 0, st_a, sem_sa)

        @pl.when(t > 0)
        def _():
            st_wait(c0 - 1, 1, st_b, sem_sb)

        do_group(tt_a, 1, st_b)
        st_start(c0, 1, st_b, sem_sb)

        tt_wait(c0 + 1, tt_b, sem_tb)
        tt_start(c0 + 2, tt_a, sem_ta)
        st_wait(c0, 0, st_a, sem_sa)
        do_group(tt_b, 0, st_a)
        st_start(c0 + 1, 0, st_a, sem_sa)
        st_wait(c0, 1, st_b, sem_sb)
        do_group(tt_b, 1, st_b)
        st_start(c0 + 1, 1, st_b, sem_sb)
        return carry

    lax.fori_loop(0, (_NCG - 1) // 2, pair_body, 0)

    cg = _NCG - 1
    tt_wait(cg, tt_a, sem_ta)
    st_wait(cg - 1, 0, st_a, sem_sa)
    do_group(tt_a, 0, st_a)
    st_start(cg, 0, st_a, sem_sa)
    st_wait(cg - 1, 1, st_b, sem_sb)
    do_group(tt_a, 1, st_b)
    st_start(cg, 1, st_b, sem_sb)
    st_wait(cg, 0, st_a, sem_sa)
    st_wait(cg, 1, st_b, sem_sb)


def kernel(idx, targets, token_emb_table):
    tableT_flat = token_emb_table.T.reshape(_V * _V)
    idx_f = idx.reshape(_N)
    tgt_f = targets.reshape(_N)
    lse = _lse_call(token_emb_table).reshape(_V)
    y4, partials = _sc_main(tableT_flat, idx_f, tgt_f, lse)
    logits2 = y4.transpose(0, 2, 1, 3).reshape(_V, _N).T
    loss = _loss_call(partials)[0, 0]
    return (logits2, loss)


# speedup vs baseline: 1.8096x; 1.0092x over previous
import functools

import jax
import jax.numpy as jnp
from jax import lax
from jax.experimental import pallas as pl
from jax.experimental.pallas import tpu as pltpu
from jax.experimental.pallas import tpu_sc as plsc

_V = 1000
_N = 204800
_NC, _NS = 2, 16
_NW = _NC * _NS
_PER_W = _N // _NW
_NCG = _V // 8
_NTI = _N // 128
_TPW = _NTI // _NW
_HT = _TPW // 2


def _lse_body(t_ref, o_ref):
    t = t_ref[...]
    m = jnp.max(t, axis=1, keepdims=True)
    o_ref[...] = m + jnp.log(jnp.sum(jnp.exp(t - m), axis=1, keepdims=True))


_lse_call = pl.pallas_call(
    _lse_body,
    out_shape=jax.ShapeDtypeStruct((_V, 1), jnp.float32),
)


def _loss_body(p_ref, o_ref):
    o_ref[0, 0] = jnp.sum(p_ref[...]) * (1.0 / _N)


_loss_call = pl.pallas_call(
    _loss_body,
    out_shape=jax.ShapeDtypeStruct((1, 1), jnp.float32),
    out_specs=pl.BlockSpec(memory_space=pltpu.SMEM),
)


_mesh = plsc.VectorSubcoreMesh(core_axis_name="c", subcore_axis_name="s")


@functools.partial(
    pl.kernel,
    out_type=(
        jax.ShapeDtypeStruct((_NCG, _NTI, 8, 128), jnp.float32),
        jax.ShapeDtypeStruct((_NW, 16), jnp.float32),
    ),
    mesh=_mesh,
    compiler_params=pltpu.CompilerParams(
        use_tc_tiling_on_sc=False, needs_layout_passes=False),
    scratch_types=[
        pltpu.VMEM((_PER_W,), jnp.int32),
        pltpu.VMEM((_PER_W,), jnp.int32),
        pltpu.VMEM((_PER_W,), jnp.float32),
        pltpu.VMEM((_V,), jnp.float32),
        pltpu.VMEM((8 * _V,), jnp.float32),
        pltpu.VMEM((8 * _V,), jnp.float32),
        pltpu.VMEM((_HT, 8, 128), jnp.float32),
        pltpu.VMEM((_HT, 8, 128), jnp.float32),
        pltpu.VMEM((16,), jnp.float32),
        pltpu.SemaphoreType.DMA,
        pltpu.SemaphoreType.DMA,
        pltpu.SemaphoreType.DMA,
        pltpu.SemaphoreType.DMA,
        pltpu.SemaphoreType.DMA,
    ],
)
def _sc_main(tt_hbm, idx_hbm, tgt_hbm, lse_hbm, out_hbm, part_hbm,
             idx_v, off_v, picked_v, lse_v, tt_a, tt_b, st_a, st_b,
             acc_v, sem_ta, sem_tb, sem_sa, sem_sb, sem_p):
    wid = lax.axis_index("s") * _NC + lax.axis_index("c")
    ebase = wid * _PER_W
    tbase = wid * _TPW

    pltpu.async_copy(tt_hbm.at[pl.ds(0, 8 * _V)], tt_a, sem_ta)
    pltpu.sync_copy(idx_hbm.at[pl.ds(ebase, _PER_W)], idx_v)
    pltpu.sync_copy(tgt_hbm.at[pl.ds(ebase, _PER_W)], off_v)
    pltpu.sync_copy(lse_hbm, lse_v)
    acc_v[...] = jnp.zeros((16,), jnp.float32)

    def off_body(g, carry):
        sl = pl.ds(g * 16, 16)
        off_v[sl] = off_v[sl] * _V + idx_v[sl]
        return carry

    lax.fori_loop(0, _PER_W // 16, off_body, 0)

    for c in range(_PER_W // 128):
        pltpu.async_copy(
            tt_hbm.at[off_v.at[pl.ds(c * 128, 128)]],
            picked_v.at[pl.ds(c * 128, 128)],
            sem_p,
        )

    def tt_start(cg, tt, sem):
        pltpu.async_copy(tt_hbm.at[pl.ds(cg * (8 * _V), 8 * _V)], tt, sem)

    def tt_wait(cg, tt, sem):
        pltpu.make_async_copy(
            tt_hbm.at[pl.ds(cg * (8 * _V), 8 * _V)], tt, sem).wait()

    def st_start(cg, h, st, sem):
        pltpu.async_copy(st, out_hbm.at[cg, pl.ds(tbase + h * _HT, _HT)], sem)

    def st_wait(cg, h, st, sem):
        pltpu.make_async_copy(
            st, out_hbm.at[cg, pl.ds(tbase + h * _HT, _HT)], sem).wait()

    def do_group(tt, h, st):
        @plsc.parallel_loop(0, _HT * 8, unroll=4)
        def _(t):
            ip = t // 8
            m = t % 8
            iv = idx_v[pl.ds(h * (_HT * 128) + t * 16, 16)]
            for s in range(8):
                st[ip, s, pl.ds(m * 16, 16)] = plsc.load_gather(
                    tt, [iv + (s * _V)])

    def pair_body(t, carry):
        c0 = 2 * t

        tt_wait(c0, tt_a, sem_ta)
        tt_start(c0 + 1, tt_b, sem_tb)

        @pl.when(t > 0)
        def _():
            st_wait(c0 - 1, 0, st_a, sem_sa)

        do_group(tt_a, 0, st_a)
        st_start(c0, 0, st_a, sem_sa)

        @pl.when(t > 0)
        def _():
            st_wait(c0 - 1, 1, st_b, sem_sb)

        do_group(tt_a, 1, st_b)
        st_start(c0, 1, st_b, sem_sb)

        tt_wait(c0 + 1, tt_b, sem_tb)
        tt_start(c0 + 2, tt_a, sem_ta)
        st_wait(c0, 0, st_a, sem_sa)
        do_group(tt_b, 0, st_a)
        st_start(c0 + 1, 0, st_a, sem_sa)
        st_wait(c0, 1, st_b, sem_sb)
        do_group(tt_b, 1, st_b)
        st_start(c0 + 1, 1, st_b, sem_sb)
        return carry

    lax.fori_loop(0, (_NCG - 1) // 2, pair_body, 0)

    cg = _NCG - 1
    tt_wait(cg, tt_a, sem_ta)
    st_wait(cg - 1, 0, st_a, sem_sa)
    do_group(tt_a, 0, st_a)
    st_start(cg, 0, st_a, sem_sa)
    st_wait(cg - 1, 1, st_b, sem_sb)
    do_group(tt_a, 1, st_b)
    st_start(cg, 1, st_b, sem_sb)

    for c in range(_PER_W // 128):
        pltpu.make_async_copy(
            tt_hbm.at[off_v.at[pl.ds(c * 128, 128)]],
            picked_v.at[pl.ds(c * 128, 128)],
            sem_p,
        ).wait()

    def lp_body(g, carry):
        sl = pl.ds(g * 16, 16)
        lseg = plsc.load_gather(lse_v, [idx_v[sl]])
        acc_v[...] = acc_v[...] + (lseg - picked_v[sl])
        return carry

    lax.fori_loop(0, _PER_W // 16, lp_body, 0)
    pltpu.sync_copy(acc_v, part_hbm.at[wid])

    st_wait(cg, 0, st_a, sem_sa)
    st_wait(cg, 1, st_b, sem_sb)


def kernel(idx, targets, token_emb_table):
    tableT_flat = token_emb_table.T.reshape(_V * _V)
    idx_f = idx.reshape(_N)
    tgt_f = targets.reshape(_N)
    lse = _lse_call(token_emb_table).reshape(_V)
    y4, partials = _sc_main(tableT_flat, idx_f, tgt_f, lse)
    logits2 = y4.transpose(0, 2, 1, 3).reshape(_V, _N).T
    loss = _loss_call(partials)[0, 0]
    return (logits2, loss)


# speedup vs baseline: 1.8109x; 1.0007x over previous
import functools

import jax
import jax.numpy as jnp
from jax import lax
from jax.experimental import pallas as pl
from jax.experimental.pallas import tpu as pltpu
from jax.experimental.pallas import tpu_sc as plsc

_V = 1000
_N = 204800
_NC, _NS = 2, 16
_NW = _NC * _NS
_PER_W = _N // _NW
_NCG = _V // 8
_NTI = _N // 128
_TPW = _NTI // _NW
_HT = _TPW // 2


def _lse_body(t_ref, o_ref):
    t = t_ref[...]
    m = jnp.max(t, axis=1, keepdims=True)
    o_ref[...] = m + jnp.log(jnp.sum(jnp.exp(t - m), axis=1, keepdims=True))


_lse_call = pl.pallas_call(
    _lse_body,
    out_shape=jax.ShapeDtypeStruct((_V, 1), jnp.float32),
)


def _loss_body(p_ref, o_ref):
    o_ref[0, 0] = jnp.sum(p_ref[...]) * (1.0 / _N)


_loss_call = pl.pallas_call(
    _loss_body,
    out_shape=jax.ShapeDtypeStruct((1, 1), jnp.float32),
    out_specs=pl.BlockSpec(memory_space=pltpu.SMEM),
)


_mesh = plsc.VectorSubcoreMesh(core_axis_name="c", subcore_axis_name="s")


@functools.partial(
    pl.kernel,
    out_type=(
        jax.ShapeDtypeStruct((_NCG, _NTI, 8, 128), jnp.float32),
        jax.ShapeDtypeStruct((_NW, 16), jnp.float32),
    ),
    mesh=_mesh,
    compiler_params=pltpu.CompilerParams(
        use_tc_tiling_on_sc=False, needs_layout_passes=False),
    scratch_types=[
        pltpu.VMEM((_PER_W,), jnp.int32),
        pltpu.VMEM((_PER_W,), jnp.int32),
        pltpu.VMEM((_PER_W,), jnp.float32),
        pltpu.VMEM((_V,), jnp.float32),
        pltpu.VMEM((8 * _V,), jnp.float32),
        pltpu.VMEM((8 * _V,), jnp.float32),
        pltpu.VMEM((_HT, 8, 128), jnp.float32),
        pltpu.VMEM((_HT, 8, 128), jnp.float32),
        pltpu.VMEM((16,), jnp.float32),
        pltpu.SemaphoreType.DMA,
        pltpu.SemaphoreType.DMA,
        pltpu.SemaphoreType.DMA,
        pltpu.SemaphoreType.DMA,
        pltpu.SemaphoreType.DMA,
    ],
)
def _sc_main(tt_hbm, idx_hbm, tgt_hbm, lse_hbm, out_hbm, part_hbm,
             idx_v, off_v, picked_v, lse_v, tt_a, tt_b, st_a, st_b,
             acc_v, sem_ta, sem_tb, sem_sa, sem_sb, sem_p):
    wid = lax.axis_index("s") * _NC + lax.axis_index("c")
    ebase = wid * _PER_W
    tbase = wid * _TPW

    pltpu.async_copy(tt_hbm.at[pl.ds(0, 8 * _V)], tt_a, sem_ta)
    pltpu.sync_copy(idx_hbm.at[pl.ds(ebase, _PER_W)], idx_v)
    pltpu.sync_copy(tgt_hbm.at[pl.ds(ebase, _PER_W)], off_v)
    pltpu.sync_copy(lse_hbm, lse_v)
    acc_v[...] = jnp.zeros((16,), jnp.float32)

    def off_body(g, carry):
        sl = pl.ds(g * 16, 16)
        off_v[sl] = off_v[sl] * _V + idx_v[sl]
        return carry

    lax.fori_loop(0, _PER_W // 16, off_body, 0)

    for c in range(_PER_W // 128):
        pltpu.async_copy(
            tt_hbm.at[off_v.at[pl.ds(c * 128, 128)]],
            picked_v.at[pl.ds(c * 128, 128)],
            sem_p,
        )

    def tt_start(cg, tt, sem):
        pltpu.async_copy(tt_hbm.at[pl.ds(cg * (8 * _V), 8 * _V)], tt, sem)

    def tt_wait(cg, tt, sem):
        pltpu.make_async_copy(
            tt_hbm.at[pl.ds(cg * (8 * _V), 8 * _V)], tt, sem).wait()

    def st_start(cg, h, st, sem):
        pltpu.async_copy(st, out_hbm.at[cg, pl.ds(tbase + h * _HT, _HT)], sem)

    def st_wait(cg, h, st, sem):
        pltpu.make_async_copy(
            st, out_hbm.at[cg, pl.ds(tbase + h * _HT, _HT)], sem).wait()

    def do_group(tt, h, st):
        @plsc.parallel_loop(0, _HT * 8, unroll=8)
        def _(t):
            ip = t // 8
            m = t % 8
            iv = idx_v[pl.ds(h * (_HT * 128) + t * 16, 16)]
            for s in range(8):
                st[ip, s, pl.ds(m * 16, 16)] = plsc.load_gather(
                    tt, [iv + (s * _V)])

    def pair_body(t, carry):
        c0 = 2 * t

        tt_wait(c0, tt_a, sem_ta)
        tt_start(c0 + 1, tt_b, sem_tb)

        @pl.when(t > 0)
        def _():
            st_wait(c0 - 1, 0, st_a, sem_sa)

        do_group(tt_a, 0, st_a)
        st_start(c0, 0, st_a, sem_sa)

        @pl.when(t > 0)
        def _():
            st_wait(c0 - 1, 1, st_b, sem_sb)

        do_group(tt_a, 1, st_b)
        st_start(c0, 1, st_b, sem_sb)

        tt_wait(c0 + 1, tt_b, sem_tb)
        tt_start(c0 + 2, tt_a, sem_ta)
        st_wait(c0, 0, st_a, sem_sa)
        do_group(tt_b, 0, st_a)
        st_start(c0 + 1, 0, st_a, sem_sa)
        st_wait(c0, 1, st_b, sem_sb)
        do_group(tt_b, 1, st_b)
        st_start(c0 + 1, 1, st_b, sem_sb)
        return carry

    lax.fori_loop(0, (_NCG - 1) // 2, pair_body, 0)

    cg = _NCG - 1
    tt_wait(cg, tt_a, sem_ta)
    st_wait(cg - 1, 0, st_a, sem_sa)
    do_group(tt_a, 0, st_a)
    st_start(cg, 0, st_a, sem_sa)
    st_wait(cg - 1, 1, st_b, sem_sb)
    do_group(tt_a, 1, st_b)
    st_start(cg, 1, st_b, sem_sb)

    for c in range(_PER_W // 128):
        pltpu.make_async_copy(
            tt_hbm.at[off_v.at[pl.ds(c * 128, 128)]],
            picked_v.at[pl.ds(c * 128, 128)],
            sem_p,
        ).wait()

    def lp_body(g, carry):
        sl = pl.ds(g * 16, 16)
        lseg = plsc.load_gather(lse_v, [idx_v[sl]])
        acc_v[...] = acc_v[...] + (lseg - picked_v[sl])
        return carry

    lax.fori_loop(0, _PER_W // 16, lp_body, 0)
    pltpu.sync_copy(acc_v, part_hbm.at[wid])

    st_wait(cg, 0, st_a, sem_sa)
    st_wait(cg, 1, st_b, sem_sb)


def kernel(idx, targets, token_emb_table):
    tableT_flat = token_emb_table.T.reshape(_V * _V)
    idx_f = idx.reshape(_N)
    tgt_f = targets.reshape(_N)
    lse = _lse_call(token_emb_table).reshape(_V)
    y4, partials = _sc_main(tableT_flat, idx_f, tgt_f, lse)
    logits2 = y4.transpose(0, 2, 1, 3).reshape(_V, _N).T
    loss = _loss_call(partials)[0, 0]
    return (logits2, loss)
